# Initial kernel scaffold; baseline (speedup 1.0000x reference)
import jax
import jax.numpy as jnp
from jax.experimental import pallas as pl


def kernel(indices, table):
    raise NotImplementedError("write your pallas kernel here")


# speedup vs baseline: 1.5549x; 1.5549x over previous
import functools

import jax
import jax.numpy as jnp
from jax import lax
from jax.experimental import pallas as pl
from jax.experimental.pallas import tpu as pltpu
from jax.experimental.pallas import tpu_sc as plsc

N_ROWS = 16384 * 26
D = 32

_info = plsc.get_sparse_core_info()
_NC, _NS = _info.num_cores, _info.num_subcores
_NW = _NC * _NS
_B_PER_W = N_ROWS // _NW
_CHUNK = 1024
_N_CHUNKS = _B_PER_W // _CHUNK

_mesh = plsc.VectorSubcoreMesh(core_axis_name="c", subcore_axis_name="s")


@functools.partial(
    pl.kernel,
    mesh=_mesh,
    compiler_params=pltpu.CompilerParams(use_tc_tiling_on_sc=False),
    out_type=jax.ShapeDtypeStruct((N_ROWS, D), jnp.float32),
    scratch_types=[
        pltpu.VMEM((_CHUNK,), jnp.int32),
        pltpu.VMEM((_CHUNK, D), jnp.float32),
        pltpu.SemaphoreType.DMA,
    ],
)
def _gather_sc(idx_hbm, table_hbm, out_hbm, idx_v, rows_v, sem):
    wid = lax.axis_index("s") * _NC + lax.axis_index("c")
    base = pl.multiple_of(wid * _B_PER_W, _CHUNK)

    def body(g, carry):
        start = pl.multiple_of(base + g * _CHUNK, _CHUNK)
        pltpu.sync_copy(idx_hbm.at[pl.ds(start, _CHUNK)], idx_v)
        pltpu.async_copy(table_hbm.at[idx_v], rows_v, sem).wait()
        pltpu.sync_copy(rows_v, out_hbm.at[pl.ds(start, _CHUNK)])
        return carry

    lax.fori_loop(0, _N_CHUNKS, body, 0)


def kernel(indices, table):
    idx_flat = indices.reshape(-1).astype(jnp.int32)
    out = _gather_sc(idx_flat, table)
    return out.reshape(indices.shape + (D,))


# speedup vs baseline: 1.5744x; 1.0125x over previous
import functools

import jax
import jax.numpy as jnp
from jax import lax
from jax.experimental import pallas as pl
from jax.experimental.pallas import tpu as pltpu
from jax.experimental.pallas import tpu_sc as plsc

N_ROWS = 16384 * 26
D = 32

_info = plsc.get_sparse_core_info()
_NC, _NS = _info.num_cores, _info.num_subcores
_NW = _NC * _NS
_B_PER_W = N_ROWS // _NW
_N_CHUNKS = 8
_CHUNK = _B_PER_W // _N_CHUNKS

_mesh = plsc.VectorSubcoreMesh(core_axis_name="c", subcore_axis_name="s")


@functools.partial(
    pl.kernel,
    mesh=_mesh,
    compiler_params=pltpu.CompilerParams(use_tc_tiling_on_sc=False),
    out_type=jax.ShapeDtypeStruct((N_ROWS, D), jnp.float32),
    scratch_types=[
        pltpu.VMEM((_N_CHUNKS, _CHUNK), jnp.int32),
        pltpu.VMEM((2, _CHUNK, D), jnp.float32),
        pltpu.SemaphoreType.DMA,
        pltpu.SemaphoreType.DMA,
        pltpu.SemaphoreType.DMA,
        pltpu.SemaphoreType.DMA,
    ],
)
def _gather_sc(idx_hbm, table_hbm, out_hbm, idx_all, rows, gs0, gs1, ws0, ws1):
    wid = lax.axis_index("s") * _NC + lax.axis_index("c")
    rbase = wid * _B_PER_W
    pltpu.sync_copy(idx_hbm.at[pl.ds(wid * _N_CHUNKS, _N_CHUNKS)], idx_all)

    gsem = (gs0, gs1)
    wsem = (ws0, ws1)
    gh = [None] * _N_CHUNKS
    wh = [None] * _N_CHUNKS
    for g in range(_N_CHUNKS):
        b = g % 2
        if g >= 2:
            wh[g - 2].wait()
        gh[g] = pltpu.async_copy(table_hbm.at[idx_all.at[g]], rows.at[b], gsem[b])
        gh[g].wait()
        wh[g] = pltpu.async_copy(
            rows.at[b], out_hbm.at[pl.ds(rbase + g * _CHUNK, _CHUNK)], wsem[b]
        )
    wh[_N_CHUNKS - 2].wait()
    wh[_N_CHUNKS - 1].wait()


def kernel(indices, table):
    idx_2d = indices.reshape(_NW * _N_CHUNKS, _CHUNK).astype(jnp.int32)
    out = _gather_sc(idx_2d, table)
    return out.reshape(indices.shape + (D,))


# speedup vs baseline: 1.9095x; 1.2128x over previous
import functools

import jax
import jax.numpy as jnp
from jax import lax
from jax.experimental import pallas as pl
from jax.experimental.pallas import tpu as pltpu
from jax.experimental.pallas import tpu_sc as plsc

N_VOC = 1000000
D = 32
N_B = 16384
N_F = 26

_info = plsc.get_sparse_core_info()
_NC, _NS = _info.num_cores, _info.num_subcores
_NW = _NC * _NS
_B_PER_W = N_B // _NW

_mesh = plsc.VectorSubcoreMesh(core_axis_name="c", subcore_axis_name="s")


_T_COLS = 2048
_T_GRID = -(-N_VOC // _T_COLS)
_T_ROWS = _T_GRID * (_T_COLS // 4)


def _t1_body(x_ref, o_ref):
    x = x_ref[...]
    o_ref[...] = jnp.concatenate(
        [x[:, q * 512:(q + 1) * 512].T for q in range(4)], axis=1
    )


_stage1 = pl.pallas_call(
    _t1_body,
    grid=(_T_GRID,),
    in_specs=[pl.BlockSpec((D, _T_COLS), lambda i: (0, i))],
    out_specs=pl.BlockSpec((_T_COLS // 4, 128), lambda i: (i, 0)),
    out_shape=jax.ShapeDtypeStruct((_T_ROWS, 128), jnp.float32),
)


@functools.partial(
    pl.kernel,
    mesh=_mesh,
    compiler_params=pltpu.CompilerParams(use_tc_tiling_on_sc=False),
    out_type=jax.ShapeDtypeStruct((N_F, N_B // 4, 128), jnp.float32),
    scratch_types=[
        pltpu.VMEM((N_F, _B_PER_W), jnp.int32),
        pltpu.VMEM((2, _B_PER_W, D), jnp.float32),
        pltpu.SemaphoreType.DMA,
        pltpu.SemaphoreType.DMA,
        pltpu.SemaphoreType.DMA,
        pltpu.SemaphoreType.DMA,
    ],
)
def _stage2(idx_hbm, table_hbm, out_hbm, idx_all, rows, gs0, gs1, ws0, ws1):
    wid = lax.axis_index("s") * _NC + lax.axis_index("c")
    b0 = wid * _B_PER_W
    prow = (wid // 4) * _B_PER_W
    pcol = (wid % 4) * D
    pltpu.sync_copy(idx_hbm.at[:, pl.ds(b0, _B_PER_W)], idx_all)

    gsem = (gs0, gs1)
    wsem = (ws0, ws1)
    gh = [None] * N_F
    wh = [None] * N_F
    for f in range(N_F):
        b = f % 2
        if f >= 2:
            wh[f - 2].wait()
        gh[f] = pltpu.async_copy(table_hbm.at[idx_all.at[f]], rows.at[b], gsem[b])
        gh[f].wait()
        wh[f] = pltpu.async_copy(
            rows.at[b],
            out_hbm.at[f, pl.ds(prow, _B_PER_W), pl.ds(pcol, D)],
            wsem[b],
        )
    wh[N_F - 2].wait()
    wh[N_F - 1].wait()


def _t3_body(x_ref, o_ref):
    x = x_ref[0]
    o_ref[0] = jnp.concatenate(
        [x[:, q * D:(q + 1) * D].T for q in range(4)], axis=1
    )


_stage3 = pl.pallas_call(
    _t3_body,
    grid=(N_F, N_B // _T_COLS),
    in_specs=[pl.BlockSpec((1, 512, 128), lambda f, k: (f, k, 0))],
    out_specs=pl.BlockSpec((1, D, _T_COLS), lambda f, k: (f, 0, k)),
    out_shape=jax.ShapeDtypeStruct((N_F, D, N_B), jnp.float32),
)


def kernel(indices, table):
    idx = indices.T.astype(jnp.int32)
    blk = idx >> 11
    q = (idx >> 9) & 3
    r = idx & 511
    idx_m = ((blk << 9) + r) * 4 + q
    table_w = _stage1(table.T)
    table_l = table_w.reshape(_T_ROWS * 4, D)
    planes = _stage2(idx_m, table_l)
    out3 = _stage3(planes)
    return out3.transpose(2, 0, 1)


# speedup vs baseline: 2.5632x; 1.3424x over previous
import functools

import jax
import jax.numpy as jnp
from jax import lax
from jax.experimental import pallas as pl
from jax.experimental.pallas import tpu as pltpu
from jax.experimental.pallas import tpu_sc as plsc

N_VOC = 1000000
D = 32
N_B = 16384
N_F = 26

_info = plsc.get_sparse_core_info()
_NC, _NS = _info.num_cores, _info.num_subcores
_NW = _NC * _NS
_B_PER_W = N_B // _NW

_mesh = plsc.VectorSubcoreMesh(core_axis_name="c", subcore_axis_name="s")


_T_COLS = 16384
_T_GRID = -(-N_VOC // _T_COLS)
_T_ROWS = _T_GRID * (_T_COLS // 4)


_T_Q = _T_COLS // 4


def _t1_body(x_ref, o_ref):
    x = x_ref[...]
    o_ref[...] = jnp.concatenate(
        [x[:, q * _T_Q:(q + 1) * _T_Q].T for q in range(4)], axis=1
    )


_stage1 = pl.pallas_call(
    _t1_body,
    grid=(_T_GRID,),
    in_specs=[pl.BlockSpec((D, _T_COLS), lambda i: (0, i))],
    out_specs=pl.BlockSpec((_T_COLS // 4, 128), lambda i: (i, 0)),
    out_shape=jax.ShapeDtypeStruct((_T_ROWS, 128), jnp.float32),
)


@functools.partial(
    pl.kernel,
    mesh=_mesh,
    compiler_params=pltpu.CompilerParams(use_tc_tiling_on_sc=False),
    out_type=jax.ShapeDtypeStruct((N_F, N_B // 4, 128), jnp.float32),
    scratch_types=[
        pltpu.VMEM((N_F, _B_PER_W), jnp.int32),
        pltpu.VMEM((2, _B_PER_W, D), jnp.float32),
        pltpu.SemaphoreType.DMA,
        pltpu.SemaphoreType.DMA,
        pltpu.SemaphoreType.DMA,
        pltpu.SemaphoreType.DMA,
    ],
)
def _stage2(idx_hbm, table_hbm, out_hbm, idx_all, rows, gs0, gs1, ws0, ws1):
    wid = lax.axis_index("s") * _NC + lax.axis_index("c")
    b0 = wid * _B_PER_W
    prow = (wid // 4) * _B_PER_W
    pcol = (wid % 4) * D
    pltpu.sync_copy(idx_hbm.at[:, pl.ds(b0, _B_PER_W)], idx_all)

    gsem = (gs0, gs1)
    wsem = (ws0, ws1)
    gh = [None] * N_F
    wh = [None] * N_F
    for f in range(N_F):
        b = f % 2
        if f >= 2:
            wh[f - 2].wait()
        gh[f] = pltpu.async_copy(table_hbm.at[idx_all.at[f]], rows.at[b], gsem[b])
        gh[f].wait()
        wh[f] = pltpu.async_copy(
            rows.at[b],
            out_hbm.at[f, pl.ds(prow, _B_PER_W), pl.ds(pcol, D)],
            wsem[b],
        )
    wh[N_F - 2].wait()
    wh[N_F - 1].wait()


def _t3_body(x_ref, o_ref):
    x = x_ref[0]
    o_ref[0] = jnp.concatenate(
        [x[:, q * D:(q + 1) * D].T for q in range(4)], axis=1
    )


_stage3 = pl.pallas_call(
    _t3_body,
    grid=(N_F, N_B // 2048),
    in_specs=[pl.BlockSpec((1, 512, 128), lambda f, k: (f, k, 0))],
    out_specs=pl.BlockSpec((1, D, 2048), lambda f, k: (f, 0, k)),
    out_shape=jax.ShapeDtypeStruct((N_F, D, N_B), jnp.float32),
)


def kernel(indices, table):
    idx = indices.T.astype(jnp.int32)
    blk = idx >> 14
    q = (idx >> 12) & 3
    r = idx & 4095
    idx_m = ((blk << 12) + r) * 4 + q
    table_w = _stage1(table.T)
    table_l = table_w.reshape(_T_ROWS * 4, D)
    planes = _stage2(idx_m, table_l)
    out3 = _stage3(planes)
    return out3.transpose(2, 0, 1)


# speedup vs baseline: 3.0081x; 1.1736x over previous
import functools

import jax
import jax.numpy as jnp
from jax import lax
from jax.experimental import pallas as pl
from jax.experimental.pallas import tpu as pltpu
from jax.experimental.pallas import tpu_sc as plsc

N_VOC = 1000000
D = 32
N_B = 16384
N_F = 26

_info = plsc.get_sparse_core_info()
_NC, _NS = _info.num_cores, _info.num_subcores
_NW = _NC * _NS
_B_PER_W = N_B // _NW

_mesh = plsc.VectorSubcoreMesh(core_axis_name="c", subcore_axis_name="s")


_T_COLS = 32768
_T_GRID = -(-N_VOC // _T_COLS)
_T_ROWS = _T_GRID * (_T_COLS // 4)


_T_Q = _T_COLS // 4


def _t1_body(x_ref, o_ref):
    x = x_ref[...]
    o_ref[...] = jnp.concatenate(
        [x[:, q * _T_Q:(q + 1) * _T_Q].T for q in range(4)], axis=1
    )


_stage1 = pl.pallas_call(
    _t1_body,
    grid=(_T_GRID,),
    in_specs=[pl.BlockSpec((D, _T_COLS), lambda i: (0, i))],
    out_specs=pl.BlockSpec((_T_COLS // 4, 128), lambda i: (i, 0)),
    out_shape=jax.ShapeDtypeStruct((_T_ROWS, 128), jnp.float32),
)


@functools.partial(
    pl.kernel,
    mesh=_mesh,
    compiler_params=pltpu.CompilerParams(use_tc_tiling_on_sc=False),
    out_type=jax.ShapeDtypeStruct((N_F, N_B // 4, 128), jnp.float32),
    scratch_types=[
        pltpu.VMEM((N_F, _B_PER_W), jnp.int32),
        pltpu.VMEM((2, _B_PER_W, D), jnp.float32),
        pltpu.SemaphoreType.DMA,
        pltpu.SemaphoreType.DMA,
        pltpu.SemaphoreType.DMA,
        pltpu.SemaphoreType.DMA,
    ],
)
def _stage2(idx_hbm, table_hbm, out_hbm, idx_all, rows, gs0, gs1, ws0, ws1):
    wid = lax.axis_index("s") * _NC + lax.axis_index("c")
    b0 = wid * _B_PER_W
    prow = (wid % 8) * _B_PER_W
    pcol = (wid // 8) * D
    pltpu.sync_copy(idx_hbm.at[:, pl.ds(b0, _B_PER_W)], idx_all)

    gsem = (gs0, gs1)
    wsem = (ws0, ws1)
    gh = [None] * N_F
    wh = [None] * N_F
    for f in range(N_F):
        b = f % 2
        if f >= 2:
            wh[f - 2].wait()
        gh[f] = pltpu.async_copy(table_hbm.at[idx_all.at[f]], rows.at[b], gsem[b])
        gh[f].wait()
        wh[f] = pltpu.async_copy(
            rows.at[b],
            out_hbm.at[f, pl.ds(prow, _B_PER_W), pl.ds(pcol, D)],
            wsem[b],
        )
    wh[N_F - 2].wait()
    wh[N_F - 1].wait()


def _t3_body(x_ref, o_ref):
    x = x_ref[0]
    o_ref[0] = jnp.concatenate(
        [x[:, q * D:(q + 1) * D].T for q in range(4)], axis=1
    )


_stage3 = pl.pallas_call(
    _t3_body,
    grid=(N_F,),
    in_specs=[pl.BlockSpec((1, N_B // 4, 128), lambda f: (f, 0, 0))],
    out_specs=pl.BlockSpec((1, D, N_B), lambda f: (f, 0, 0)),
    out_shape=jax.ShapeDtypeStruct((N_F, D, N_B), jnp.float32),
)


def kernel(indices, table):
    idx = indices.T.astype(jnp.int32)
    blk = idx >> 15
    q = (idx >> 13) & 3
    r = idx & 8191
    idx_m = ((blk << 13) + r) * 4 + q
    table_w = _stage1(table.T)
    table_l = table_w.reshape(_T_ROWS * 4, D)
    planes = _stage2(idx_m, table_l)
    out3 = _stage3(planes)
    return out3.transpose(2, 0, 1)
